# Initial kernel scaffold; baseline (speedup 1.0000x reference)
import jax
import jax.numpy as jnp
from jax.experimental import pallas as pl


def kernel(x, edge_index, batch_idx, W1, b1, W2, b2, Wh, bh):
    raise NotImplementedError("write your pallas kernel here")


# speedup vs baseline: 11.4490x; 11.4490x over previous
import functools

import jax
import jax.numpy as jnp
from jax import lax
from jax.experimental import pallas as pl
from jax.experimental.pallas import tpu as pltpu, tpu_sc as plsc

N = 10000
NP = 10240
E = 320000
D = 128
NG = 64
DEGW = 16

NC = 2
NS = 16
NW = NC * NS
EPW = E // NW
GD = 80
NCHD = EPW // GD
G = 128
EPWP = 10240
NCH = EPWP // G
PH = NCH // 2
RPS = NP // NS

_mesh = plsc.VectorSubcoreMesh(core_axis_name="c", subcore_axis_name="s")


@functools.partial(
    pl.kernel,
    out_type=jax.ShapeDtypeStruct((NC, NP, DEGW), jnp.float32),
    mesh=_mesh,
    scratch_types=[
        pltpu.VMEM((NCHD, GD), jnp.int32),
        pltpu.VMEM((GD, DEGW), jnp.float32),
        pltpu.VMEM_SHARED((NP, DEGW), jnp.float32),
    ],
)
def _sc_degree(ei_hbm, ones_hbm, z16_hbm, out_hbm, didx, obuf, acc):
    cid = lax.axis_index("c")
    sid = lax.axis_index("s")
    w = cid * NS + sid
    pltpu.sync_copy(ei_hbm.at[1, w], didx)
    pltpu.sync_copy(ones_hbm, obuf)
    pltpu.sync_copy(z16_hbm, acc.at[pl.ds(sid * RPS, RPS)])
    plsc.subcore_barrier()

    def body(j, carry):
        pltpu.sync_copy(obuf, acc.at[didx.at[j]], add=True)
        return carry

    lax.fori_loop(0, NCHD, body, 0)
    plsc.subcore_barrier()
    pltpu.sync_copy(acc.at[pl.ds(sid * RPS, RPS)],
                    out_hbm.at[cid, pl.ds(sid * RPS, RPS)])


@functools.partial(
    pl.kernel,
    out_type=jax.ShapeDtypeStruct((NC, NP, D), jnp.float32),
    mesh=_mesh,
    scratch_types=[
        pltpu.VMEM((2 * PH, G), jnp.int32),
        pltpu.VMEM((G, D), jnp.float32),
        pltpu.VMEM((G, D), jnp.float32),
        pltpu.VMEM_SHARED((NP, D), jnp.float32),
        pltpu.SemaphoreType.DMA,
        pltpu.SemaphoreType.DMA,
    ],
)
def _sc_propagate(h_hbm, ei_hbm, z_hbm, out_hbm,
                  ibuf, rbuf0, rbuf1, acc, gsem0, gsem1):
    cid = lax.axis_index("c")
    sid = lax.axis_index("s")
    w = cid * NS + sid
    pltpu.sync_copy(z_hbm, acc.at[pl.ds(sid * RPS, RPS)])
    plsc.subcore_barrier()

    for p in range(2):
        pltpu.sync_copy(ei_hbm.at[w, pl.ds(p * 2 * PH, 2 * PH)], ibuf)
        pltpu.async_copy(h_hbm.at[ibuf.at[0]], rbuf0, gsem0)
        pltpu.async_copy(h_hbm.at[ibuf.at[2]], rbuf1, gsem1)

        def body(i, carry):
            j = i * 2
            for b, (rb, sem) in enumerate(((rbuf0, gsem0), (rbuf1, gsem1))):
                jj = j + b
                pltpu.make_async_copy(h_hbm.at[pl.ds(0, G)], rb, sem).wait()
                pltpu.sync_copy(rb, acc.at[ibuf.at[2 * jj + 1]], add=True)

                @pl.when(jj + 2 < PH)
                def _():
                    pltpu.async_copy(h_hbm.at[ibuf.at[2 * (jj + 2)]], rb, sem)
            return carry

        lax.fori_loop(0, PH // 2, body, 0)

    plsc.subcore_barrier()
    pltpu.sync_copy(acc.at[pl.ds(sid * RPS, RPS)],
                    out_hbm.at[cid, pl.ds(sid * RPS, RPS)])


def _tc_prep_body(cnt_ref, x_ref, w1_ref, h1p_ref, disb_ref):
    c = cnt_ref[...]
    deg = c[0, :N, 0:1] + c[1, :N, 0:1] + 1.0
    dis = lax.rsqrt(deg)
    h = jnp.dot(x_ref[...], w1_ref[...], preferred_element_type=jnp.float32)
    h1p_ref[...] = h * dis
    disb_ref[...] = jnp.broadcast_to(dis, (N, D))


def _tc_mid_body(agg_ref, h1p_ref, disb_ref, b1_ref, w2_ref, h2p_ref):
    agg = agg_ref[0, :N] + agg_ref[1, :N]
    z = jnp.maximum(disb_ref[...] * (agg + h1p_ref[...]) + b1_ref[...], 0.0)
    h2p_ref[...] = jnp.dot(z, w2_ref[...],
                           preferred_element_type=jnp.float32) * disb_ref[...]


def _tc_final_body(agg_ref, h2p_ref, disb_ref, b2_ref, bidx_ref, whp_ref,
                   bhp_ref, out_ref):
    agg = agg_ref[0, :N] + agg_ref[1, :N]
    g = disb_ref[...] * (agg + h2p_ref[...]) + b2_ref[...]
    gid = lax.broadcasted_iota(jnp.int32, (NG, N), 0)
    p = (gid == bidx_ref[...]).astype(jnp.float32)
    sums = jnp.dot(p, g, preferred_element_type=jnp.float32)
    cnt = jnp.sum(p, axis=1, keepdims=True)
    pooled = sums / jnp.maximum(cnt, 1.0)
    out_ref[...] = jnp.dot(pooled, whp_ref[...],
                           preferred_element_type=jnp.float32) + bhp_ref[...]


_tc_prep = pl.pallas_call(
    _tc_prep_body,
    out_shape=(jax.ShapeDtypeStruct((N, D), jnp.float32),
               jax.ShapeDtypeStruct((N, D), jnp.float32)),
)

_tc_mid = pl.pallas_call(
    _tc_mid_body,
    out_shape=jax.ShapeDtypeStruct((N, D), jnp.float32),
)

_tc_final = pl.pallas_call(
    _tc_final_body,
    out_shape=jax.ShapeDtypeStruct((NG, D), jnp.float32),
)


def kernel(x, edge_index, batch_idx, W1, b1, W2, b2, Wh, bh):
    ei_deg = edge_index.reshape(2, NW, NCHD, GD)
    src_p = jnp.pad(edge_index[0].reshape(NW, EPW), ((0, 0), (0, EPWP - EPW)),
                    constant_values=0).reshape(NW, NCH, 1, G)
    dst_p = jnp.pad(edge_index[1].reshape(NW, EPW), ((0, 0), (0, EPWP - EPW)),
                    constant_values=N).reshape(NW, NCH, 1, G)
    ei = jnp.concatenate([src_p, dst_p], axis=2).reshape(NW, 2 * NCH, G)
    ones16 = jnp.ones((GD, DEGW), jnp.float32)
    z16 = jnp.zeros((RPS, DEGW), jnp.float32)
    z = jnp.zeros((RPS, D), jnp.float32)
    bidx2 = batch_idx.reshape(1, N)
    whp = jnp.pad(Wh, ((0, 0), (0, D - Wh.shape[1])))
    bhp = jnp.pad(bh, (0, D - bh.shape[0]))

    cnt = _sc_degree(ei_deg, ones16, z16)
    h1p, disb = _tc_prep(cnt, x, W1)
    agg1 = _sc_propagate(h1p, ei, z)
    h2p = _tc_mid(agg1, h1p, disb, b1, W2)
    agg2 = _sc_propagate(h2p, ei, z)
    out = _tc_final(agg2, h2p, disb, b2, bidx2, whp, bhp)
    return out[:, :Wh.shape[1]]
